# baseline (speedup 1.0000x reference)
import functools

import jax
import jax.numpy as jnp
from jax import lax
from jax.experimental import pallas as pl
from jax.experimental.pallas import tpu as pltpu
from jax.experimental.pallas import tpu_sc as plsc

E = 8
TOP = 2
EMB = 384
HID = 2 * EMB
B = 16
HW = 1024
N_GATES = 4


def _merge_body(W1_ref, W2_ref, b1_ref, b2_ref, gamma_ref, beta_ref, rm_ref,
                rv_ref, W3_ref, W12T_ref, b12r_ref, W3T_ref):
    s_r = gamma_ref[0] * lax.rsqrt(rv_ref[0] + 1e-5)
    t_r = beta_ref[0] - rm_ref[0] * s_r
    b12_r = jax.lax.dot_general(b1_ref[0], W2_ref[0], (((1,), (1,)), ((), ())),
                                preferred_element_type=jnp.float32) + b2_ref[0]
    b12r_ref[0] = s_r * b12_r + t_r
    w12t = jax.lax.dot_general(W1_ref[0], W2_ref[0], (((0,), (1,)), ((), ())),
                               preferred_element_type=jnp.float32)
    W12T_ref[0] = w12t * s_r
    ii = lax.broadcasted_iota(jnp.int32, (HID, HID), 0)
    jj = lax.broadcasted_iota(jnp.int32, (HID, HID), 1)
    eye = (ii == jj).astype(jnp.float32)
    W3T_ref[0] = jax.lax.dot_general(eye, W3_ref[0], (((1,), (1,)), ((), ())),
                                     preferred_element_type=jnp.float32)


def _router_body(x_ref, gates_ref, lg_ref):
    pooled = jnp.sum(x_ref[...], axis=1) * (1.0 / HW)
    for g in range(N_GATES):
        lg_ref[g] = jax.lax.dot_general(gates_ref[g], pooled,
                                        (((0,), (1,)), ((), ())),
                                        preferred_element_type=jnp.float32)


def _route_sc_body(lg_hbm, w_hbm, cnt_hbm, se_hbm, lg_v, w_v, cnt_v, se_v):
    c = lax.axis_index("c")
    s = lax.axis_index("s")

    @pl.when(jnp.logical_and(c == 0, s == 0))
    def _():
        pltpu.sync_copy(lg_hbm, lg_v)
        iota = lax.iota(jnp.int32, 16)
        neginf = jnp.full((16,), -jnp.inf, jnp.float32)
        one = jnp.full((16,), 1, jnp.int32)
        zero = jnp.full((16,), 0, jnp.int32)
        for k in range(E):
            se_v[pl.ds(k * 16, 16)] = zero
        u = [zero for _ in range(E)]
        for g in range(N_GATES):
            l = [lg_v[g, e, :] for e in range(E)]
            m = l[0]
            for e in range(1, E):
                m = jnp.maximum(m, l[e])
            ex = [jnp.exp(l[e] - m) for e in range(E)]
            z = ex[0]
            for e in range(1, E):
                z = z + ex[e]
            p = [ex[e] / z for e in range(E)]
            m1 = p[0]
            for e in range(1, E):
                m1 = jnp.maximum(m1, p[e])
            i1 = jnp.full((16,), E, jnp.int32)
            for e in range(E - 1, -1, -1):
                i1 = jnp.where(p[e] == m1, jnp.full((16,), e, jnp.int32), i1)
            p2 = [jnp.where(i1 == e, neginf, p[e]) for e in range(E)]
            m2 = p2[0]
            for e in range(1, E):
                m2 = jnp.maximum(m2, p2[e])
            i2 = jnp.full((16,), E, jnp.int32)
            for e in range(E - 1, -1, -1):
                i2 = jnp.where(p2[e] == m2, jnp.full((16,), e, jnp.int32), i2)
            e1 = jnp.exp(m1)
            e2 = jnp.exp(m2)
            w1 = e1 / (e1 + e2)
            w2 = e2 / (e1 + e2)
            for e in range(E):
                sel1 = i1 == e
                sel2 = i2 == e
                we = jnp.where(sel1, w1, jnp.where(sel2, w2, 0.0))
                w_v[g, e, :] = we
                u[e] = jnp.maximum(u[e], jnp.where(sel1 | sel2, one, zero))
        run = zero
        for e in range(E):
            mask = u[e] > 0
            plsc.store_scatter(se_v, [iota * E + run],
                               jnp.full((16,), e, jnp.int32), mask=mask)
            run = run + u[e]
        cnt_v[...] = run
        pltpu.sync_copy(w_v, w_hbm)
        pltpu.sync_copy(cnt_v, cnt_hbm)
        pltpu.sync_copy(se_v, se_hbm)


def _main_body(cnt_ref, se_ref, w_ref, x_ref, W12T_ref, b12r_ref, W3T_ref,
               b3_ref, o0_ref, o1_ref, o2_ref, o3_ref):
    t = pl.program_id(0)
    s = pl.program_id(1)
    outs = (o0_ref, o1_ref, o2_ref, o3_ref)

    @pl.when(s == 0)
    def _():
        for o in outs:
            o[0] = jnp.zeros((HW, EMB), jnp.float32)

    @pl.when(s < cnt_ref[t])
    def _():
        e = se_ref[t, s]
        xb = x_ref[0]
        h = jnp.dot(xb, W12T_ref[e], preferred_element_type=jnp.float32)
        h = jnp.maximum(h + b12r_ref[e], 0.0)
        y = jnp.dot(h, W3T_ref[e], preferred_element_type=jnp.float32)
        y = y + b3_ref[e]
        for g in range(N_GATES):
            outs[g][0] += w_ref[g, e, t] * y


def kernel(x, gates, W1, b1, W2, b2, gamma, beta, rm, rv, W3, b3):
    xt = jnp.transpose(x, (0, 2, 3, 1)).reshape(B, HW, EMB)

    lgT = pl.pallas_call(
        _router_body,
        grid=(1,),
        in_specs=[
            pl.BlockSpec((B, HW, EMB), lambda i: (0, 0, 0)),
            pl.BlockSpec((N_GATES, EMB, E), lambda i: (0, 0, 0)),
        ],
        out_specs=pl.BlockSpec((N_GATES, E, B), lambda i: (0, 0, 0)),
        out_shape=jax.ShapeDtypeStruct((N_GATES, E, B), jnp.float32),
    )(xt, gates)

    route = pl.kernel(
        _route_sc_body,
        out_type=[
            jax.ShapeDtypeStruct((N_GATES, E, B), jnp.float32),
            jax.ShapeDtypeStruct((B,), jnp.int32),
            jax.ShapeDtypeStruct((B * E,), jnp.int32),
        ],
        mesh=plsc.VectorSubcoreMesh(core_axis_name="c", subcore_axis_name="s",
                                    num_cores=2, num_subcores=16),
        scratch_types=[
            pltpu.VMEM((N_GATES, E, B), jnp.float32),
            pltpu.VMEM((N_GATES, E, B), jnp.float32),
            pltpu.VMEM((B,), jnp.int32),
            pltpu.VMEM((B * E,), jnp.int32),
        ],
        compiler_params=pltpu.CompilerParams(needs_layout_passes=False),
    )
    w, cnt, se_flat = route(lgT)
    se = se_flat.reshape(B, E)

    merged = pl.pallas_call(
        _merge_body,
        grid=(E,),
        in_specs=[
            pl.BlockSpec((1, HID, EMB), lambda e: (e, 0, 0)),
            pl.BlockSpec((1, HID, HID), lambda e: (e, 0, 0)),
            pl.BlockSpec((1, 1, HID), lambda e: (e, 0, 0)),
            pl.BlockSpec((1, 1, HID), lambda e: (e, 0, 0)),
            pl.BlockSpec((1, 1, HID), lambda e: (e, 0, 0)),
            pl.BlockSpec((1, 1, HID), lambda e: (e, 0, 0)),
            pl.BlockSpec((1, 1, HID), lambda e: (e, 0, 0)),
            pl.BlockSpec((1, 1, HID), lambda e: (e, 0, 0)),
            pl.BlockSpec((1, EMB, HID), lambda e: (e, 0, 0)),
        ],
        out_specs=[
            pl.BlockSpec((1, EMB, HID), lambda e: (e, 0, 0)),
            pl.BlockSpec((1, 1, HID), lambda e: (e, 0, 0)),
            pl.BlockSpec((1, HID, EMB), lambda e: (e, 0, 0)),
        ],
        out_shape=[
            jax.ShapeDtypeStruct((E, EMB, HID), jnp.float32),
            jax.ShapeDtypeStruct((E, 1, HID), jnp.float32),
            jax.ShapeDtypeStruct((E, HID, EMB), jnp.float32),
        ],
    )(W1, W2, b1.reshape(E, 1, HID), b2.reshape(E, 1, HID),
      gamma.reshape(E, 1, HID), beta.reshape(E, 1, HID),
      rm.reshape(E, 1, HID), rv.reshape(E, 1, HID), W3)
    W12T, b12r, W3T = merged

    grid_spec = pltpu.PrefetchScalarGridSpec(
        num_scalar_prefetch=3,
        grid=(B, E),
        in_specs=[
            pl.BlockSpec((1, HW, EMB), lambda t, s, c, e, w_: (t, 0, 0)),
            pl.BlockSpec((E, EMB, HID), lambda t, s, c, e, w_: (0, 0, 0)),
            pl.BlockSpec((E, 1, HID), lambda t, s, c, e, w_: (0, 0, 0)),
            pl.BlockSpec((E, HID, EMB), lambda t, s, c, e, w_: (0, 0, 0)),
            pl.BlockSpec((E, 1, EMB), lambda t, s, c, e, w_: (0, 0, 0)),
        ],
        out_specs=[
            pl.BlockSpec((1, HW, EMB), lambda t, s, c, e, w_: (t, 0, 0))
            for _ in range(N_GATES)
        ],
    )
    outs = pl.pallas_call(
        _main_body,
        grid_spec=grid_spec,
        out_shape=[jax.ShapeDtypeStruct((B, HW, EMB), jnp.float32)
                   for _ in range(N_GATES)],
        compiler_params=pltpu.CompilerParams(
            dimension_semantics=("arbitrary", "arbitrary"),
        ),
    )(cnt, se, w, xt, W12T, b12r, W3T, b3.reshape(E, 1, EMB))

    return tuple(
        o.reshape(B, 32, 32, EMB).transpose(0, 3, 1, 2) for o in outs)


# speedup vs baseline: 1.0182x; 1.0182x over previous
import functools

import jax
import jax.numpy as jnp
from jax import lax
from jax.experimental import pallas as pl
from jax.experimental.pallas import tpu as pltpu
from jax.experimental.pallas import tpu_sc as plsc

E = 8
TOP = 2
EMB = 384
HID = 2 * EMB
B = 16
HW = 1024
N_GATES = 4


def _merge_body(W1_ref, W2_ref, b1_ref, b2_ref, gamma_ref, beta_ref, rm_ref,
                rv_ref, W3_ref, W12T_ref, b12r_ref, W3T_ref):
    s_r = gamma_ref[0] * lax.rsqrt(rv_ref[0] + 1e-5)
    t_r = beta_ref[0] - rm_ref[0] * s_r
    b12_r = jax.lax.dot_general(b1_ref[0], W2_ref[0], (((1,), (1,)), ((), ())),
                                preferred_element_type=jnp.float32) + b2_ref[0]
    b12r_ref[0] = s_r * b12_r + t_r
    w12t = jax.lax.dot_general(W1_ref[0], W2_ref[0], (((0,), (1,)), ((), ())),
                               preferred_element_type=jnp.float32)
    W12T_ref[0] = w12t * s_r
    ii = lax.broadcasted_iota(jnp.int32, (HID, HID), 0)
    jj = lax.broadcasted_iota(jnp.int32, (HID, HID), 1)
    eye = (ii == jj).astype(jnp.float32)
    W3T_ref[0] = jax.lax.dot_general(eye, W3_ref[0], (((1,), (1,)), ((), ())),
                                     preferred_element_type=jnp.float32)


def _router_body(x_ref, gates_ref, lg_ref):
    pooled = jnp.sum(x_ref[...], axis=1) * (1.0 / HW)
    for g in range(N_GATES):
        lg_ref[g] = jax.lax.dot_general(gates_ref[g], pooled,
                                        (((0,), (1,)), ((), ())),
                                        preferred_element_type=jnp.float32)


def _route_sc_body(lg_hbm, w_hbm, cnt_hbm, se_hbm, lg_v, w_v, cnt_v, se_v):
    c = lax.axis_index("c")
    s = lax.axis_index("s")

    @pl.when(jnp.logical_and(c == 0, s == 0))
    def _():
        pltpu.sync_copy(lg_hbm, lg_v)
        iota = lax.iota(jnp.int32, 16)
        neginf = jnp.full((16,), -jnp.inf, jnp.float32)
        one = jnp.full((16,), 1, jnp.int32)
        zero = jnp.full((16,), 0, jnp.int32)
        for k in range(E):
            se_v[pl.ds(k * 16, 16)] = zero
        u = [zero for _ in range(E)]
        for g in range(N_GATES):
            l = [lg_v[g, e, :] for e in range(E)]
            m = l[0]
            for e in range(1, E):
                m = jnp.maximum(m, l[e])
            ex = [jnp.exp(l[e] - m) for e in range(E)]
            z = ex[0]
            for e in range(1, E):
                z = z + ex[e]
            p = [ex[e] / z for e in range(E)]
            m1 = p[0]
            for e in range(1, E):
                m1 = jnp.maximum(m1, p[e])
            i1 = jnp.full((16,), E, jnp.int32)
            for e in range(E - 1, -1, -1):
                i1 = jnp.where(p[e] == m1, jnp.full((16,), e, jnp.int32), i1)
            p2 = [jnp.where(i1 == e, neginf, p[e]) for e in range(E)]
            m2 = p2[0]
            for e in range(1, E):
                m2 = jnp.maximum(m2, p2[e])
            i2 = jnp.full((16,), E, jnp.int32)
            for e in range(E - 1, -1, -1):
                i2 = jnp.where(p2[e] == m2, jnp.full((16,), e, jnp.int32), i2)
            e1 = jnp.exp(m1)
            e2 = jnp.exp(m2)
            w1 = e1 / (e1 + e2)
            w2 = e2 / (e1 + e2)
            for e in range(E):
                sel1 = i1 == e
                sel2 = i2 == e
                we = jnp.where(sel1, w1, jnp.where(sel2, w2, 0.0))
                w_v[g, e, :] = we
                u[e] = jnp.maximum(u[e], jnp.where(sel1 | sel2, one, zero))
        run = zero
        for e in range(E):
            mask = u[e] > 0
            plsc.store_scatter(se_v, [iota * E + run],
                               jnp.full((16,), e, jnp.int32), mask=mask)
            run = run + u[e]
        cnt_v[...] = run
        pltpu.sync_copy(w_v, w_hbm)
        pltpu.sync_copy(cnt_v, cnt_hbm)
        pltpu.sync_copy(se_v, se_hbm)


def _main_body(cnt_ref, se_ref, w_ref, x_ref, W12T_ref, b12r_ref, W3T_ref,
               b3_ref, o0_ref, o1_ref, o2_ref, o3_ref):
    t = pl.program_id(0)
    s = pl.program_id(1)
    outs = (o0_ref, o1_ref, o2_ref, o3_ref)

    @pl.when(s == 0)
    def _():
        for o in outs:
            o[0] = jnp.zeros((HW, EMB), jnp.float32)

    @pl.when(s < cnt_ref[t])
    def _():
        e = se_ref[t, s]
        xb = x_ref[0]
        h = jnp.dot(xb, W12T_ref[e], preferred_element_type=jnp.float32)
        h = jnp.maximum(h + b12r_ref[e], 0.0)
        y = jnp.dot(h, W3T_ref[e], preferred_element_type=jnp.float32)
        y = y + b3_ref[e]
        for g in range(N_GATES):
            wg = w_ref[g, e, t]

            @pl.when(wg != 0.0)
            def _():
                outs[g][0] += wg * y


def kernel(x, gates, W1, b1, W2, b2, gamma, beta, rm, rv, W3, b3):
    xt = jnp.transpose(x, (0, 2, 3, 1)).reshape(B, HW, EMB)

    lgT = pl.pallas_call(
        _router_body,
        grid=(1,),
        in_specs=[
            pl.BlockSpec((B, HW, EMB), lambda i: (0, 0, 0)),
            pl.BlockSpec((N_GATES, EMB, E), lambda i: (0, 0, 0)),
        ],
        out_specs=pl.BlockSpec((N_GATES, E, B), lambda i: (0, 0, 0)),
        out_shape=jax.ShapeDtypeStruct((N_GATES, E, B), jnp.float32),
    )(xt, gates)

    route = pl.kernel(
        _route_sc_body,
        out_type=[
            jax.ShapeDtypeStruct((N_GATES, E, B), jnp.float32),
            jax.ShapeDtypeStruct((B,), jnp.int32),
            jax.ShapeDtypeStruct((B * E,), jnp.int32),
        ],
        mesh=plsc.VectorSubcoreMesh(core_axis_name="c", subcore_axis_name="s",
                                    num_cores=2, num_subcores=16),
        scratch_types=[
            pltpu.VMEM((N_GATES, E, B), jnp.float32),
            pltpu.VMEM((N_GATES, E, B), jnp.float32),
            pltpu.VMEM((B,), jnp.int32),
            pltpu.VMEM((B * E,), jnp.int32),
        ],
        compiler_params=pltpu.CompilerParams(needs_layout_passes=False),
    )
    w, cnt, se_flat = route(lgT)
    se = se_flat.reshape(B, E)

    merged = pl.pallas_call(
        _merge_body,
        grid=(E,),
        in_specs=[
            pl.BlockSpec((1, HID, EMB), lambda e: (e, 0, 0)),
            pl.BlockSpec((1, HID, HID), lambda e: (e, 0, 0)),
            pl.BlockSpec((1, 1, HID), lambda e: (e, 0, 0)),
            pl.BlockSpec((1, 1, HID), lambda e: (e, 0, 0)),
            pl.BlockSpec((1, 1, HID), lambda e: (e, 0, 0)),
            pl.BlockSpec((1, 1, HID), lambda e: (e, 0, 0)),
            pl.BlockSpec((1, 1, HID), lambda e: (e, 0, 0)),
            pl.BlockSpec((1, 1, HID), lambda e: (e, 0, 0)),
            pl.BlockSpec((1, EMB, HID), lambda e: (e, 0, 0)),
        ],
        out_specs=[
            pl.BlockSpec((1, EMB, HID), lambda e: (e, 0, 0)),
            pl.BlockSpec((1, 1, HID), lambda e: (e, 0, 0)),
            pl.BlockSpec((1, HID, EMB), lambda e: (e, 0, 0)),
        ],
        out_shape=[
            jax.ShapeDtypeStruct((E, EMB, HID), jnp.float32),
            jax.ShapeDtypeStruct((E, 1, HID), jnp.float32),
            jax.ShapeDtypeStruct((E, HID, EMB), jnp.float32),
        ],
    )(W1, W2, b1.reshape(E, 1, HID), b2.reshape(E, 1, HID),
      gamma.reshape(E, 1, HID), beta.reshape(E, 1, HID),
      rm.reshape(E, 1, HID), rv.reshape(E, 1, HID), W3)
    W12T, b12r, W3T = merged

    grid_spec = pltpu.PrefetchScalarGridSpec(
        num_scalar_prefetch=3,
        grid=(B, E),
        in_specs=[
            pl.BlockSpec((1, HW, EMB), lambda t, s, c, e, w_: (t, 0, 0)),
            pl.BlockSpec((E, EMB, HID), lambda t, s, c, e, w_: (0, 0, 0)),
            pl.BlockSpec((E, 1, HID), lambda t, s, c, e, w_: (0, 0, 0)),
            pl.BlockSpec((E, HID, EMB), lambda t, s, c, e, w_: (0, 0, 0)),
            pl.BlockSpec((E, 1, EMB), lambda t, s, c, e, w_: (0, 0, 0)),
        ],
        out_specs=[
            pl.BlockSpec((1, HW, EMB), lambda t, s, c, e, w_: (t, 0, 0))
            for _ in range(N_GATES)
        ],
    )
    outs = pl.pallas_call(
        _main_body,
        grid_spec=grid_spec,
        out_shape=[jax.ShapeDtypeStruct((B, HW, EMB), jnp.float32)
                   for _ in range(N_GATES)],
        compiler_params=pltpu.CompilerParams(
            dimension_semantics=("arbitrary", "arbitrary"),
        ),
    )(cnt, se, w, xt, W12T, b12r, W3T, b3.reshape(E, 1, EMB))

    return tuple(
        o.reshape(B, 32, 32, EMB).transpose(0, 3, 1, 2) for o in outs)


# speedup vs baseline: 1.0191x; 1.0009x over previous
import jax
import jax.numpy as jnp
from jax import lax
from jax.experimental import pallas as pl
from jax.experimental.pallas import tpu as pltpu
from jax.experimental.pallas import tpu_sc as plsc

E = 8
TOP = 2
EMB = 384
HID = 2 * EMB
B = 16
HW = 1024
N_GATES = 4


def _merge_body(W1_ref, W2_ref, b1_ref, b2_ref, gamma_ref, beta_ref, rm_ref,
                rv_ref, W3_ref, W12T_ref, b12r_ref, W3T_ref):
    s_r = gamma_ref[0] * lax.rsqrt(rv_ref[0] + 1e-5)
    t_r = beta_ref[0] - rm_ref[0] * s_r
    b12_r = jax.lax.dot_general(b1_ref[0], W2_ref[0], (((1,), (1,)), ((), ())),
                                preferred_element_type=jnp.float32) + b2_ref[0]
    b12r_ref[0] = s_r * b12_r + t_r
    w12t = jax.lax.dot_general(W1_ref[0], W2_ref[0], (((0,), (1,)), ((), ())),
                               preferred_element_type=jnp.float32)
    W12T_ref[0] = w12t * s_r
    ii = lax.broadcasted_iota(jnp.int32, (HID, HID), 0)
    jj = lax.broadcasted_iota(jnp.int32, (HID, HID), 1)
    eye = (ii == jj).astype(jnp.float32)
    W3T_ref[0] = jax.lax.dot_general(eye, W3_ref[0], (((1,), (1,)), ((), ())),
                                     preferred_element_type=jnp.float32)


def _router_body(x_ref, gates_ref, lg_ref):
    pooled = jnp.sum(x_ref[...], axis=1) * (1.0 / HW)
    for g in range(N_GATES):
        lg_ref[g] = jax.lax.dot_general(gates_ref[g], pooled,
                                        (((0,), (1,)), ((), ())),
                                        preferred_element_type=jnp.float32)


def _route_sc_body(lg_hbm, w_hbm, cnt_hbm, se_hbm, lg_v, w_v, cnt_v, se_v):
    c = lax.axis_index("c")
    s = lax.axis_index("s")

    @pl.when(jnp.logical_and(c == 0, s == 0))
    def _():
        pltpu.sync_copy(lg_hbm, lg_v)
        iota = lax.iota(jnp.int32, 16)
        neginf = jnp.full((16,), -jnp.inf, jnp.float32)
        one = jnp.full((16,), 1, jnp.int32)
        zero = jnp.full((16,), 0, jnp.int32)
        for k in range(E):
            se_v[pl.ds(k * 16, 16)] = zero
        u = [zero for _ in range(E)]
        for g in range(N_GATES):
            l = [lg_v[g, e, :] for e in range(E)]
            m = l[0]
            for e in range(1, E):
                m = jnp.maximum(m, l[e])
            ex = [jnp.exp(l[e] - m) for e in range(E)]
            z = ex[0]
            for e in range(1, E):
                z = z + ex[e]
            p = [ex[e] / z for e in range(E)]
            m1 = p[0]
            for e in range(1, E):
                m1 = jnp.maximum(m1, p[e])
            i1 = jnp.full((16,), E, jnp.int32)
            for e in range(E - 1, -1, -1):
                i1 = jnp.where(p[e] == m1, jnp.full((16,), e, jnp.int32), i1)
            p2 = [jnp.where(i1 == e, neginf, p[e]) for e in range(E)]
            m2 = p2[0]
            for e in range(1, E):
                m2 = jnp.maximum(m2, p2[e])
            i2 = jnp.full((16,), E, jnp.int32)
            for e in range(E - 1, -1, -1):
                i2 = jnp.where(p2[e] == m2, jnp.full((16,), e, jnp.int32), i2)
            e1 = jnp.exp(m1)
            e2 = jnp.exp(m2)
            w1 = e1 / (e1 + e2)
            w2 = e2 / (e1 + e2)
            for e in range(E):
                sel1 = i1 == e
                sel2 = i2 == e
                we = jnp.where(sel1, w1, jnp.where(sel2, w2, 0.0))
                w_v[g, e, :] = we
                u[e] = jnp.maximum(u[e], jnp.where(sel1 | sel2, one, zero))
        run = zero
        for e in range(E):
            mask = u[e] > 0
            plsc.store_scatter(se_v, [iota * E + run],
                               jnp.full((16,), e, jnp.int32), mask=mask)
            run = run + u[e]
        cnt_v[...] = run
        pltpu.sync_copy(w_v, w_hbm)
        pltpu.sync_copy(cnt_v, cnt_hbm)
        pltpu.sync_copy(se_v, se_hbm)


def _main_body(cnt_ref, se_ref, w_ref, x_ref, W12T_ref, b12r_ref, W3T_ref,
               b3_ref, o0_ref, o1_ref, o2_ref, o3_ref):
    t = pl.program_id(0)
    s = pl.program_id(1)
    outs = (o0_ref, o1_ref, o2_ref, o3_ref)

    @pl.when(s == 0)
    def _():
        for o in outs:
            o[0] = jnp.zeros((HW, EMB), jnp.float32)

    @pl.when(s < cnt_ref[t])
    def _():
        e = se_ref[t, s]
        xb = x_ref[0]
        h = jnp.dot(xb, W12T_ref[e], preferred_element_type=jnp.float32)
        h = jnp.maximum(h + b12r_ref[e], 0.0)
        y = jnp.dot(h, W3T_ref[e], preferred_element_type=jnp.float32)
        y = y + b3_ref[e]
        for g in range(N_GATES):
            wg = w_ref[g, e, t]

            @pl.when(wg != 0.0)
            def _():
                outs[g][0] += wg * y


def kernel(x, gates, W1, b1, W2, b2, gamma, beta, rm, rv, W3, b3):
    xt = jnp.transpose(x, (0, 2, 3, 1)).reshape(B, HW, EMB)

    lgT = pl.pallas_call(
        _router_body,
        grid=(1,),
        in_specs=[
            pl.BlockSpec((B, HW, EMB), lambda i: (0, 0, 0)),
            pl.BlockSpec((N_GATES, EMB, E), lambda i: (0, 0, 0)),
        ],
        out_specs=pl.BlockSpec((N_GATES, E, B), lambda i: (0, 0, 0)),
        out_shape=jax.ShapeDtypeStruct((N_GATES, E, B), jnp.float32),
    )(xt, gates)

    route = pl.kernel(
        _route_sc_body,
        out_type=[
            jax.ShapeDtypeStruct((N_GATES, E, B), jnp.float32),
            jax.ShapeDtypeStruct((B,), jnp.int32),
            jax.ShapeDtypeStruct((B * E,), jnp.int32),
        ],
        mesh=plsc.VectorSubcoreMesh(core_axis_name="c", subcore_axis_name="s",
                                    num_cores=2, num_subcores=16),
        scratch_types=[
            pltpu.VMEM((N_GATES, E, B), jnp.float32),
            pltpu.VMEM((N_GATES, E, B), jnp.float32),
            pltpu.VMEM((B,), jnp.int32),
            pltpu.VMEM((B * E,), jnp.int32),
        ],
        compiler_params=pltpu.CompilerParams(needs_layout_passes=False),
    )
    w, cnt, se_flat = route(lgT)
    se = se_flat.reshape(B, E)

    merged = pl.pallas_call(
        _merge_body,
        grid=(E,),
        in_specs=[
            pl.BlockSpec((1, HID, EMB), lambda e: (e, 0, 0)),
            pl.BlockSpec((1, HID, HID), lambda e: (e, 0, 0)),
            pl.BlockSpec((1, 1, HID), lambda e: (e, 0, 0)),
            pl.BlockSpec((1, 1, HID), lambda e: (e, 0, 0)),
            pl.BlockSpec((1, 1, HID), lambda e: (e, 0, 0)),
            pl.BlockSpec((1, 1, HID), lambda e: (e, 0, 0)),
            pl.BlockSpec((1, 1, HID), lambda e: (e, 0, 0)),
            pl.BlockSpec((1, 1, HID), lambda e: (e, 0, 0)),
            pl.BlockSpec((1, EMB, HID), lambda e: (e, 0, 0)),
        ],
        out_specs=[
            pl.BlockSpec((1, EMB, HID), lambda e: (e, 0, 0)),
            pl.BlockSpec((1, 1, HID), lambda e: (e, 0, 0)),
            pl.BlockSpec((1, HID, EMB), lambda e: (e, 0, 0)),
        ],
        out_shape=[
            jax.ShapeDtypeStruct((E, EMB, HID), jnp.float32),
            jax.ShapeDtypeStruct((E, 1, HID), jnp.float32),
            jax.ShapeDtypeStruct((E, HID, EMB), jnp.float32),
        ],
    )(W1, W2, b1.reshape(E, 1, HID), b2.reshape(E, 1, HID),
      gamma.reshape(E, 1, HID), beta.reshape(E, 1, HID),
      rm.reshape(E, 1, HID), rv.reshape(E, 1, HID), W3)
    W12T, b12r, W3T = merged

    grid_spec = pltpu.PrefetchScalarGridSpec(
        num_scalar_prefetch=3,
        grid=(B, E),
        in_specs=[
            pl.BlockSpec((1, HW, EMB), lambda t, s, c, e, w_: (t, 0, 0)),
            pl.BlockSpec((E, EMB, HID), lambda t, s, c, e, w_: (0, 0, 0)),
            pl.BlockSpec((E, 1, HID), lambda t, s, c, e, w_: (0, 0, 0)),
            pl.BlockSpec((E, HID, EMB), lambda t, s, c, e, w_: (0, 0, 0)),
            pl.BlockSpec((E, 1, EMB), lambda t, s, c, e, w_: (0, 0, 0)),
        ],
        out_specs=[
            pl.BlockSpec((1, HW, EMB), lambda t, s, c, e, w_: (t, 0, 0))
            for _ in range(N_GATES)
        ],
    )
    outs = pl.pallas_call(
        _main_body,
        grid_spec=grid_spec,
        out_shape=[jax.ShapeDtypeStruct((B, HW, EMB), jnp.float32)
                   for _ in range(N_GATES)],
        compiler_params=pltpu.CompilerParams(
            dimension_semantics=("arbitrary", "arbitrary"),
        ),
    )(cnt, se, w, xt, W12T, b12r, W3T, b3.reshape(E, 1, EMB))

    return tuple(
        o.reshape(B, 32, 32, EMB).transpose(0, 3, 1, 2) for o in outs)
